# Initial kernel scaffold; baseline (speedup 1.0000x reference)
import jax
import jax.numpy as jnp
from jax.experimental import pallas as pl


def kernel(x, edge_index, W_amp, b_amp, W_g1, b_g1, W_diff, b_diff, Wg1, bg1, Wg2, bg2):
    raise NotImplementedError("write your pallas kernel here")


# speedup vs baseline: 16.0609x; 16.0609x over previous
import functools

import jax
import jax.numpy as jnp
from jax import lax
from jax.experimental import pallas as pl
from jax.experimental.pallas import tpu as pltpu
from jax.experimental.pallas import tpu_sc as plsc

N_NODES = 10000
D = 128
N_CORES = 2
N_SUB = 16
NW = N_CORES * N_SUB
GROUP = 64
NPAD = 10240
ROWS_PER_TILE = NPAD // N_SUB
PAD_NODE = N_NODES

def _mesh():
    return plsc.VectorSubcoreMesh(
        core_axis_name="c", subcore_axis_name="s",
        num_cores=N_CORES, num_subcores=N_SUB)


def _hist_pass(ng):
    out_type = [jax.ShapeDtypeStruct((NW, 2, NPAD), jnp.float32)]
    scratch = [
        pltpu.VMEM((ng * GROUP,), jnp.int32),
        pltpu.VMEM((ng * GROUP,), jnp.int32),
        pltpu.VMEM((NPAD,), jnp.float32),
        pltpu.VMEM((NPAD,), jnp.float32),
    ]

    def body(didx_ref, sidx_ref, hist_out, didx_v, sidx_v, deg_loc, od_loc):
        c = lax.axis_index("c")
        s = lax.axis_index("s")
        w = c * N_SUB + s

        pltpu.sync_copy(didx_ref.at[w], didx_v)
        pltpu.sync_copy(sidx_ref.at[w], sidx_v)

        zero16 = jnp.zeros((16,), jnp.float32)

        @pl.loop(0, NPAD // 16)
        def _zero_hist(i):
            deg_loc[pl.ds(i * 16, 16)] = zero16
            od_loc[pl.ds(i * 16, 16)] = zero16

        ones16 = jnp.ones((16,), jnp.float32)

        @pl.loop(0, ng * (GROUP // 16))
        def _hist(j):
            dv = didx_v[pl.ds(j * 16, 16)]
            plsc.addupdate_scatter(deg_loc, [dv], ones16)
            sv = sidx_v[pl.ds(j * 16, 16)]
            plsc.addupdate_scatter(od_loc, [sv], ones16)

        pltpu.sync_copy(deg_loc, hist_out.at[w, 0])
        pltpu.sync_copy(od_loc, hist_out.at[w, 1])

    return pl.kernel(
        body, out_type=out_type, mesh=_mesh(), scratch_types=scratch,
        compiler_params=pltpu.CompilerParams(needs_layout_passes=False))


def _edge_pass(ng):
    out_type = [jax.ShapeDtypeStruct((N_CORES, NPAD, D), jnp.float32)]

    scratch = [
        pltpu.VMEM((ng * GROUP,), jnp.int32),
        pltpu.VMEM((ng * GROUP,), jnp.int32),
        pltpu.VMEM((GROUP, D), jnp.float32),
        pltpu.VMEM((GROUP, D), jnp.float32),
        pltpu.VMEM_SHARED((NPAD, D), jnp.float32),
        pltpu.SemaphoreType.DMA,
        pltpu.SemaphoreType.DMA,
    ]

    def body(table_ref, gidx_ref, sidx_ref, zeros_ref, acc_out,
             gidx_v, sidx_v, rows_a, rows_b, acc_sc, sem_a, sem_b):
        c = lax.axis_index("c")
        s = lax.axis_index("s")
        w = c * N_SUB + s

        pltpu.sync_copy(gidx_ref.at[w], gidx_v)
        pltpu.sync_copy(sidx_ref.at[w], sidx_v)

        pltpu.sync_copy(zeros_ref.at[pl.ds(s * ROWS_PER_TILE, ROWS_PER_TILE)],
                        acc_sc.at[pl.ds(s * ROWS_PER_TILE, ROWS_PER_TILE)])

        plsc.subcore_barrier()

        bufs = (rows_a, rows_b)
        sems = (sem_a, sem_b)

        def gslice(g):
            return gidx_v.at[pl.ds(g * GROUP, GROUP)]

        def sslice(g):
            return sidx_v.at[pl.ds(g * GROUP, GROUP)]

        pltpu.async_copy(table_ref.at[gslice(0)], rows_a, sem_a)

        @pl.loop(0, ng)
        def _main(g):
            nxt = g + 1

            @pl.when(nxt < ng)
            def _start():
                for b in range(2):
                    @pl.when(lax.rem(nxt, 2) == b)
                    def _():
                        pltpu.async_copy(table_ref.at[gslice(nxt)],
                                         bufs[b], sems[b])

            for b in range(2):
                @pl.when(lax.rem(g, 2) == b)
                def _drain():
                    pltpu.make_async_copy(table_ref.at[gslice(g)],
                                          bufs[b], sems[b]).wait()
                    pltpu.sync_copy(bufs[b], acc_sc.at[sslice(g)],
                                    add=True)

        plsc.subcore_barrier()
        pltpu.sync_copy(acc_sc.at[pl.ds(s * ROWS_PER_TILE, ROWS_PER_TILE)],
                        acc_out.at[c, pl.ds(s * ROWS_PER_TILE, ROWS_PER_TILE)])

    return pl.kernel(
        body, out_type=out_type, mesh=_mesh(), scratch_types=scratch,
        compiler_params=pltpu.CompilerParams(needs_layout_passes=False))


def _prep_edges(idx, ng):
    total = NW * ng * GROUP
    pad = total - idx.shape[0]
    idx = jnp.concatenate(
        [idx, jnp.full((pad,), PAD_NODE, jnp.int32)])
    return idx.reshape(NW, ng * GROUP)


def _tc_norm(hist, x_pad, blk):
    grid = NPAD // blk

    def body(hist_ref, x_ref, dinv_ref, od_ref, y_ref):
        hsum = jnp.sum(hist_ref[...], axis=0)
        dinv = lax.rsqrt(hsum[0:1, :] + 1.0)
        od = hsum[1:2, :]
        dinv_col = jnp.transpose(dinv)
        od_col = jnp.transpose(od)
        dinv_ref[...] = dinv_col
        od_ref[...] = od_col
        y_ref[...] = dinv_col * x_ref[...]

    return pl.pallas_call(
        body,
        grid=(grid,),
        in_specs=[
            pl.BlockSpec((NW, 2, blk), lambda i: (0, 0, i)),
            pl.BlockSpec((blk, D), lambda i: (i, 0)),
        ],
        out_specs=[
            pl.BlockSpec((blk, 1), lambda i: (i, 0)),
            pl.BlockSpec((blk, 1), lambda i: (i, 0)),
            pl.BlockSpec((blk, D), lambda i: (i, 0)),
        ],
        out_shape=[
            jax.ShapeDtypeStruct((NPAD, 1), jnp.float32),
            jax.ShapeDtypeStruct((NPAD, 1), jnp.float32),
            jax.ShapeDtypeStruct((NPAD, D), jnp.float32),
        ],
    )(hist, x_pad)


def _tc_fuse(x_pad, U2, S2, dinv, odcol, W_amp, b_amp, W_g1, b_g1,
             W_diff, b_diff, W1a, W1b, bg1, w2row, bg2, blk):
    grid = NPAD // blk

    def body(x_ref, u_ref, s_ref, dinv_ref, od_ref,
             wamp_ref, bamp_ref, wg1_ref, bg1c_ref, wdiff_ref, bdiff_ref,
             w1a_ref, w1b_ref, bgate_ref, w2_ref, bg2_ref, out_ref):
        xb = x_ref[...]
        ub = u_ref[0] + u_ref[1]
        sb = s_ref[0] + s_ref[1]
        db = dinv_ref[...]
        od = od_ref[...]

        P = db * ub + (db * db) * xb
        h_align = jnp.dot(P, wamp_ref[...],
                          preferred_element_type=jnp.float32) + bamp_ref[...]
        h = jax.nn.relu(jnp.dot(P, wg1_ref[...],
                                preferred_element_type=jnp.float32)
                        + bg1c_ref[...])
        cm = jnp.dot(xb * sb, wdiff_ref[...],
                     preferred_element_type=jnp.float32) \
            + od * bdiff_ref[...]
        h_div = h + cm
        g = jax.nn.relu(
            jnp.dot(h_align, w1a_ref[...], preferred_element_type=jnp.float32)
            + jnp.dot(h_div, w1b_ref[...], preferred_element_type=jnp.float32)
            + bgate_ref[...])
        alpha = jax.nn.sigmoid(
            jnp.sum(g * w2_ref[...], axis=1, keepdims=True) + bg2_ref[...])
        out_ref[...] = alpha * h_align + (1.0 - alpha) * h_div

    wspec = pl.BlockSpec((D, D), lambda i: (0, 0))
    bspec = pl.BlockSpec((1, D), lambda i: (0, 0))
    return pl.pallas_call(
        body,
        grid=(grid,),
        in_specs=[
            pl.BlockSpec((blk, D), lambda i: (i, 0)),
            pl.BlockSpec((2, blk, D), lambda i: (0, i, 0)),
            pl.BlockSpec((2, blk, D), lambda i: (0, i, 0)),
            pl.BlockSpec((blk, 1), lambda i: (i, 0)),
            pl.BlockSpec((blk, 1), lambda i: (i, 0)),
            wspec, bspec, wspec, bspec, wspec, bspec,
            wspec, wspec, bspec, bspec,
            pl.BlockSpec((1, 1), lambda i: (0, 0)),
        ],
        out_specs=pl.BlockSpec((blk, D), lambda i: (i, 0)),
        out_shape=jax.ShapeDtypeStruct((NPAD, D), jnp.float32),
    )(x_pad, U2, S2, dinv, odcol, W_amp, b_amp, W_g1, b_g1, W_diff, b_diff,
      W1a, W1b, bg1, w2row, bg2)


def kernel(x, edge_index, W_amp, b_amp, W_g1, b_g1, W_diff, b_diff,
           Wg1, bg1, Wg2, bg2):
    n, d = x.shape
    e = edge_index.shape[1]
    src = edge_index[0].astype(jnp.int32)
    dst = edge_index[1].astype(jnp.int32)

    ng = -(-e // (NW * GROUP))
    if ng % 2:
        ng += 1

    x_pad = jnp.zeros((NPAD, D), jnp.float32).at[:n].set(x)
    zeros = jnp.zeros((NPAD, D), jnp.float32)
    dst_prep = _prep_edges(dst, ng)
    src_prep = _prep_edges(src, ng)

    (hist,) = _hist_pass(ng)(dst_prep, src_prep)

    (s_parts,) = _edge_pass(ng)(x_pad, dst_prep, src_prep, zeros)

    dinv, odcol, y_pad = _tc_norm(hist, x_pad, 2048)

    (u_parts,) = _edge_pass(ng)(y_pad, src_prep, dst_prep, zeros)

    out_pad = _tc_fuse(
        x_pad, u_parts, s_parts, dinv, odcol,
        W_amp, b_amp.reshape(1, D), W_g1, b_g1.reshape(1, D),
        W_diff, b_diff.reshape(1, D),
        Wg1[:D], Wg1[D:], bg1.reshape(1, D),
        Wg2.reshape(1, D), bg2.reshape(1, 1), 2048)
    return out_pad[:n]


# speedup vs baseline: 19.9756x; 1.2437x over previous
import functools

import jax
import jax.numpy as jnp
from jax import lax
from jax.experimental import pallas as pl
from jax.experimental.pallas import tpu as pltpu
from jax.experimental.pallas import tpu_sc as plsc

N_NODES = 10000
D = 128
N_CORES = 2
N_SUB = 16
NW = N_CORES * N_SUB
GROUP = 128
NPAD = 10240
ROWS_PER_TILE = NPAD // N_SUB
PAD_NODE = N_NODES

def _mesh():
    return plsc.VectorSubcoreMesh(
        core_axis_name="c", subcore_axis_name="s",
        num_cores=N_CORES, num_subcores=N_SUB)


def _hist_pass(lh):
    out_type = [jax.ShapeDtypeStruct((NW, 2, NPAD), jnp.float32)]
    scratch = [
        pltpu.VMEM((lh,), jnp.int32),
        pltpu.VMEM((lh,), jnp.int32),
        pltpu.VMEM((NPAD,), jnp.float32),
        pltpu.VMEM((NPAD,), jnp.float32),
    ]

    def body(didx_ref, sidx_ref, hist_out, didx_v, sidx_v, deg_loc, od_loc):
        c = lax.axis_index("c")
        s = lax.axis_index("s")
        w = c * N_SUB + s

        pltpu.sync_copy(didx_ref.at[w], didx_v)
        pltpu.sync_copy(sidx_ref.at[w], sidx_v)

        zero16 = jnp.zeros((16,), jnp.float32)

        @pl.loop(0, NPAD // 16)
        def _zero_hist(i):
            deg_loc[pl.ds(i * 16, 16)] = zero16
            od_loc[pl.ds(i * 16, 16)] = zero16

        ones16 = jnp.ones((16,), jnp.float32)

        @pl.loop(0, lh // 16)
        def _hist(j):
            dv = didx_v[pl.ds(j * 16, 16)]
            plsc.addupdate_scatter(deg_loc, [dv], ones16)
            sv = sidx_v[pl.ds(j * 16, 16)]
            plsc.addupdate_scatter(od_loc, [sv], ones16)

        pltpu.sync_copy(deg_loc, hist_out.at[w, 0])
        pltpu.sync_copy(od_loc, hist_out.at[w, 1])

    return pl.kernel(
        body, out_type=out_type, mesh=_mesh(), scratch_types=scratch,
        compiler_params=pltpu.CompilerParams(needs_layout_passes=False))


def _su_pass(ng):
    out_type = [jax.ShapeDtypeStruct((N_CORES, NPAD, D), jnp.float32)]

    scratch = [
        pltpu.VMEM((2, 2, GROUP), jnp.int32),
        pltpu.VMEM((GROUP, D), jnp.float32),
        pltpu.VMEM((GROUP, D), jnp.float32),
        pltpu.VMEM_SHARED((NPAD, D), jnp.float32),
        pltpu.SemaphoreType.DMA,
        pltpu.SemaphoreType.DMA,
        pltpu.SemaphoreType.DMA,
        pltpu.SemaphoreType.DMA,
    ]

    def body(table_ref, idx_ref, zeros_ref, acc_out,
             idx_buf, rows_a, rows_b, acc_sc,
             isem_a, isem_b, rsem_a, rsem_b):
        c = lax.axis_index("c")
        s = lax.axis_index("s")

        pltpu.sync_copy(zeros_ref.at[pl.ds(s * ROWS_PER_TILE, ROWS_PER_TILE)],
                        acc_sc.at[pl.ds(s * ROWS_PER_TILE, ROWS_PER_TILE)])

        plsc.subcore_barrier()

        bufs = (rows_a, rows_b)
        rsems = (rsem_a, rsem_b)
        isems = (isem_a, isem_b)

        pltpu.sync_copy(idx_ref.at[c, s, 0], idx_buf.at[0])
        pltpu.async_copy(table_ref.at[idx_buf.at[0, 0]], rows_a, rsem_a)
        pltpu.async_copy(idx_ref.at[c, s, 1], idx_buf.at[1], isem_b)

        @pl.loop(0, ng)
        def _main(g):
            for b in range(2):
                @pl.when(lax.rem(g, 2) == b)
                def _step():
                    o = 1 - b
                    nxt = g + 1

                    @pl.when(nxt < ng)
                    def _start_gather():
                        pltpu.make_async_copy(idx_ref.at[c, s, nxt],
                                              idx_buf.at[o], isems[o]).wait()
                        pltpu.async_copy(table_ref.at[idx_buf.at[o, 0]],
                                         bufs[o], rsems[o])

                    pltpu.make_async_copy(table_ref.at[idx_buf.at[b, 0]],
                                          bufs[b], rsems[b]).wait()
                    pltpu.sync_copy(bufs[b], acc_sc.at[idx_buf.at[b, 1]],
                                    add=True)

                    nxt2 = g + 2

                    @pl.when(nxt2 < ng)
                    def _prefetch_idx():
                        pltpu.async_copy(idx_ref.at[c, s, nxt2],
                                         idx_buf.at[b], isems[b])

        plsc.subcore_barrier()
        pltpu.sync_copy(acc_sc.at[pl.ds(s * ROWS_PER_TILE, ROWS_PER_TILE)],
                        acc_out.at[c, pl.ds(s * ROWS_PER_TILE, ROWS_PER_TILE)])

    return pl.kernel(
        body, out_type=out_type, mesh=_mesh(), scratch_types=scratch,
        compiler_params=pltpu.CompilerParams(needs_layout_passes=False))


def _prep_hist(idx):
    return idx.reshape(NW, idx.shape[0] // NW)


def _prep_su(src, dst, ng):
    total = N_SUB * ng * GROUP
    pad = total - src.shape[0]

    def shape(a, padval):
        a = jnp.concatenate([a, jnp.full((pad,), padval, jnp.int32)])
        return a.reshape(N_SUB, ng, GROUP)

    c0 = jnp.stack([shape(dst, PAD_NODE), shape(src, PAD_NODE)], axis=2)
    c1 = jnp.stack([shape(src + NPAD, NPAD + PAD_NODE),
                    shape(dst, PAD_NODE)], axis=2)
    return jnp.stack([c0, c1], axis=0)


def _tc_norm(hist, x_pad, blk):
    grid = NPAD // blk

    def body(hist_ref, x_ref, dinv_ref, od_ref, y_ref):
        hsum = jnp.sum(hist_ref[...], axis=0)
        dinv = lax.rsqrt(hsum[0:1, :] + 1.0)
        od = hsum[1:2, :]
        dinv_col = jnp.transpose(dinv)
        od_col = jnp.transpose(od)
        dinv_ref[...] = dinv_col
        od_ref[...] = od_col
        y_ref[...] = dinv_col * x_ref[...]

    return pl.pallas_call(
        body,
        grid=(grid,),
        in_specs=[
            pl.BlockSpec((NW, 2, blk), lambda i: (0, 0, i)),
            pl.BlockSpec((blk, D), lambda i: (i, 0)),
        ],
        out_specs=[
            pl.BlockSpec((blk, 1), lambda i: (i, 0)),
            pl.BlockSpec((blk, 1), lambda i: (i, 0)),
            pl.BlockSpec((blk, D), lambda i: (i, 0)),
        ],
        out_shape=[
            jax.ShapeDtypeStruct((NPAD, 1), jnp.float32),
            jax.ShapeDtypeStruct((NPAD, 1), jnp.float32),
            jax.ShapeDtypeStruct((NPAD, D), jnp.float32),
        ],
    )(hist, x_pad)


def _tc_fuse(x_pad, U, S, dinv, odcol, W_amp, b_amp, W_g1, b_g1,
             W_diff, b_diff, W1a, W1b, bg1, w2row, bg2, blk):
    grid = NPAD // blk

    def body(x_ref, u_ref, s_ref, dinv_ref, od_ref,
             wamp_ref, bamp_ref, wg1_ref, bg1c_ref, wdiff_ref, bdiff_ref,
             w1a_ref, w1b_ref, bgate_ref, w2_ref, bg2_ref, out_ref):
        xb = x_ref[...]
        ub = u_ref[...]
        sb = s_ref[...]
        db = dinv_ref[...]
        od = od_ref[...]

        P = db * ub + (db * db) * xb
        h_align = jnp.dot(P, wamp_ref[...],
                          preferred_element_type=jnp.float32) + bamp_ref[...]
        h = jax.nn.relu(jnp.dot(P, wg1_ref[...],
                                preferred_element_type=jnp.float32)
                        + bg1c_ref[...])
        cm = jnp.dot(xb * sb, wdiff_ref[...],
                     preferred_element_type=jnp.float32) \
            + od * bdiff_ref[...]
        h_div = h + cm
        g = jax.nn.relu(
            jnp.dot(h_align, w1a_ref[...], preferred_element_type=jnp.float32)
            + jnp.dot(h_div, w1b_ref[...], preferred_element_type=jnp.float32)
            + bgate_ref[...])
        alpha = jax.nn.sigmoid(
            jnp.sum(g * w2_ref[...], axis=1, keepdims=True) + bg2_ref[...])
        out_ref[...] = alpha * h_align + (1.0 - alpha) * h_div

    wspec = pl.BlockSpec((D, D), lambda i: (0, 0))
    bspec = pl.BlockSpec((1, D), lambda i: (0, 0))
    return pl.pallas_call(
        body,
        grid=(grid,),
        in_specs=[
            pl.BlockSpec((blk, D), lambda i: (i, 0)),
            pl.BlockSpec((blk, D), lambda i: (i, 0)),
            pl.BlockSpec((blk, D), lambda i: (i, 0)),
            pl.BlockSpec((blk, 1), lambda i: (i, 0)),
            pl.BlockSpec((blk, 1), lambda i: (i, 0)),
            wspec, bspec, wspec, bspec, wspec, bspec,
            wspec, wspec, bspec, bspec,
            pl.BlockSpec((1, 1), lambda i: (0, 0)),
        ],
        out_specs=pl.BlockSpec((blk, D), lambda i: (i, 0)),
        out_shape=jax.ShapeDtypeStruct((NPAD, D), jnp.float32),
    )(x_pad, U, S, dinv, odcol, W_amp, b_amp, W_g1, b_g1, W_diff, b_diff,
      W1a, W1b, bg1, w2row, bg2)


def kernel(x, edge_index, W_amp, b_amp, W_g1, b_g1, W_diff, b_diff,
           Wg1, bg1, Wg2, bg2):
    n, d = x.shape
    e = edge_index.shape[1]
    src = edge_index[0].astype(jnp.int32)
    dst = edge_index[1].astype(jnp.int32)

    ng = -(-e // (N_SUB * GROUP))
    if ng % 2:
        ng += 1

    x_pad = jnp.zeros((NPAD, D), jnp.float32).at[:n].set(x)
    zeros = jnp.zeros((NPAD, D), jnp.float32)

    (hist,) = _hist_pass(e // NW)(_prep_hist(dst), _prep_hist(src))

    dinv, odcol, y_pad = _tc_norm(hist, x_pad, 2048)

    table = jnp.concatenate([x_pad, y_pad], axis=0)
    (acc,) = _su_pass(ng)(table, _prep_su(src, dst, ng), zeros)

    out_pad = _tc_fuse(
        x_pad, acc[1], acc[0], dinv, odcol,
        W_amp, b_amp.reshape(1, D), W_g1, b_g1.reshape(1, D),
        W_diff, b_diff.reshape(1, D),
        Wg1[:D], Wg1[D:], bg1.reshape(1, D),
        Wg2.reshape(1, D), bg2.reshape(1, 1), 2048)
    return out_pad[:n]


# speedup vs baseline: 23.7115x; 1.1870x over previous
import functools

import jax
import jax.numpy as jnp
from jax import lax
from jax.experimental import pallas as pl
from jax.experimental.pallas import tpu as pltpu
from jax.experimental.pallas import tpu_sc as plsc

N_NODES = 10000
D = 128
N_CORES = 2
N_SUB = 16
NW = N_CORES * N_SUB
GROUP = 112
DEPTH = 3
NPAD = 10240
ROWS_PER_TILE = NPAD // N_SUB
PAD_NODE = N_NODES

def _mesh():
    return plsc.VectorSubcoreMesh(
        core_axis_name="c", subcore_axis_name="s",
        num_cores=N_CORES, num_subcores=N_SUB)


def _hist_pass(lh):
    out_type = [jax.ShapeDtypeStruct((NW, 2, NPAD), jnp.float32)]
    scratch = [
        pltpu.VMEM((lh,), jnp.int32),
        pltpu.VMEM((lh,), jnp.int32),
        pltpu.VMEM((NPAD,), jnp.float32),
        pltpu.VMEM((NPAD,), jnp.float32),
    ]

    def body(didx_ref, sidx_ref, hist_out, didx_v, sidx_v, deg_loc, od_loc):
        c = lax.axis_index("c")
        s = lax.axis_index("s")
        w = c * N_SUB + s

        pltpu.sync_copy(didx_ref.at[w], didx_v)
        pltpu.sync_copy(sidx_ref.at[w], sidx_v)

        zero16 = jnp.zeros((16,), jnp.float32)

        @pl.loop(0, NPAD // 16)
        def _zero_hist(i):
            deg_loc[pl.ds(i * 16, 16)] = zero16
            od_loc[pl.ds(i * 16, 16)] = zero16

        ones16 = jnp.ones((16,), jnp.float32)

        @pl.loop(0, lh // 16)
        def _hist(j):
            dv = didx_v[pl.ds(j * 16, 16)]
            plsc.addupdate_scatter(deg_loc, [dv], ones16)
            sv = sidx_v[pl.ds(j * 16, 16)]
            plsc.addupdate_scatter(od_loc, [sv], ones16)

        pltpu.sync_copy(deg_loc, hist_out.at[w, 0])
        pltpu.sync_copy(od_loc, hist_out.at[w, 1])

    return pl.kernel(
        body, out_type=out_type, mesh=_mesh(), scratch_types=scratch,
        compiler_params=pltpu.CompilerParams(needs_layout_passes=False))


def _su_pass(ng):
    out_type = [jax.ShapeDtypeStruct((N_CORES, NPAD, D), jnp.float32)]

    scratch = [
        pltpu.VMEM((DEPTH, 2, GROUP), jnp.int32),
    ] + [pltpu.VMEM((GROUP, D), jnp.float32) for _ in range(DEPTH)] + [
        pltpu.VMEM_SHARED((NPAD, D), jnp.float32),
    ] + [pltpu.SemaphoreType.DMA] * (2 * DEPTH)

    def body(table_ref, idx_ref, zeros_ref, acc_out, idx_buf, *rest):
        bufs = rest[:DEPTH]
        acc_sc = rest[DEPTH]
        isems = rest[DEPTH + 1:DEPTH + 1 + DEPTH]
        rsems = rest[DEPTH + 1 + DEPTH:]
        c = lax.axis_index("c")
        s = lax.axis_index("s")

        pltpu.sync_copy(zeros_ref.at[pl.ds(s * ROWS_PER_TILE, ROWS_PER_TILE)],
                        acc_sc.at[pl.ds(s * ROWS_PER_TILE, ROWS_PER_TILE)])

        plsc.subcore_barrier()

        pltpu.sync_copy(idx_ref.at[c, s, 0], idx_buf.at[0])
        pltpu.sync_copy(idx_ref.at[c, s, 1], idx_buf.at[1])
        pltpu.async_copy(table_ref.at[idx_buf.at[0, 0]], bufs[0], rsems[0])
        pltpu.async_copy(table_ref.at[idx_buf.at[1, 0]], bufs[1], rsems[1])
        pltpu.async_copy(idx_ref.at[c, s, 2], idx_buf.at[2], isems[2])

        @pl.loop(0, ng)
        def _main(g):
            for b in range(DEPTH):
                @pl.when(lax.rem(g, DEPTH) == b)
                def _step():
                    b2 = (b + 2) % DEPTH

                    pltpu.make_async_copy(table_ref.at[idx_buf.at[b, 0]],
                                          bufs[b], rsems[b]).wait()
                    pltpu.sync_copy(bufs[b], acc_sc.at[idx_buf.at[b, 1]],
                                    add=True)

                    @pl.when(g + DEPTH < ng)
                    def _prefetch_idx():
                        pltpu.async_copy(idx_ref.at[c, s, g + DEPTH],
                                         idx_buf.at[b], isems[b])

                    @pl.when(g + 2 < ng)
                    def _start_gather():
                        pltpu.make_async_copy(idx_ref.at[c, s, g + 2],
                                              idx_buf.at[b2],
                                              isems[b2]).wait()
                        pltpu.async_copy(table_ref.at[idx_buf.at[b2, 0]],
                                         bufs[b2], rsems[b2])

        plsc.subcore_barrier()
        pltpu.sync_copy(acc_sc.at[pl.ds(s * ROWS_PER_TILE, ROWS_PER_TILE)],
                        acc_out.at[c, pl.ds(s * ROWS_PER_TILE, ROWS_PER_TILE)])

    return pl.kernel(
        body, out_type=out_type, mesh=_mesh(), scratch_types=scratch,
        compiler_params=pltpu.CompilerParams(needs_layout_passes=False))


def _prep_hist(idx):
    return idx.reshape(NW, idx.shape[0] // NW)


def _prep_su(src, dst, ng):
    total = N_SUB * ng * GROUP
    pad = total - src.shape[0]

    def shape(a, padval):
        a = jnp.concatenate([a, jnp.full((pad,), padval, jnp.int32)])
        return a.reshape(N_SUB, ng, GROUP)

    c0 = jnp.stack([shape(dst, PAD_NODE), shape(src, PAD_NODE)], axis=2)
    c1 = jnp.stack([shape(src + NPAD, NPAD + PAD_NODE),
                    shape(dst, PAD_NODE)], axis=2)
    return jnp.stack([c0, c1], axis=0)


def _tc_norm(hist, x_pad, blk):
    grid = NPAD // blk

    def body(hist_ref, x_ref, dinv_ref, od_ref, y_ref):
        hsum = jnp.sum(hist_ref[...], axis=0)
        dinv = lax.rsqrt(hsum[0:1, :] + 1.0)
        od = hsum[1:2, :]
        dinv_col = jnp.transpose(dinv)
        od_col = jnp.transpose(od)
        dinv_ref[...] = dinv_col
        od_ref[...] = od_col
        y_ref[...] = dinv_col * x_ref[...]

    return pl.pallas_call(
        body,
        grid=(grid,),
        in_specs=[
            pl.BlockSpec((NW, 2, blk), lambda i: (0, 0, i)),
            pl.BlockSpec((blk, D), lambda i: (i, 0)),
        ],
        out_specs=[
            pl.BlockSpec((blk, 1), lambda i: (i, 0)),
            pl.BlockSpec((blk, 1), lambda i: (i, 0)),
            pl.BlockSpec((blk, D), lambda i: (i, 0)),
        ],
        out_shape=[
            jax.ShapeDtypeStruct((NPAD, 1), jnp.float32),
            jax.ShapeDtypeStruct((NPAD, 1), jnp.float32),
            jax.ShapeDtypeStruct((NPAD, D), jnp.float32),
        ],
    )(hist, x_pad)


def _tc_fuse(x_pad, U, S, dinv, odcol, W_amp, b_amp, W_g1, b_g1,
             W_diff, b_diff, W1a, W1b, bg1, w2row, bg2, blk):
    grid = NPAD // blk

    def body(x_ref, u_ref, s_ref, dinv_ref, od_ref,
             wamp_ref, bamp_ref, wg1_ref, bg1c_ref, wdiff_ref, bdiff_ref,
             w1a_ref, w1b_ref, bgate_ref, w2_ref, bg2_ref, out_ref):
        xb = x_ref[...]
        ub = u_ref[...]
        sb = s_ref[...]
        db = dinv_ref[...]
        od = od_ref[...]

        P = db * ub + (db * db) * xb
        h_align = jnp.dot(P, wamp_ref[...],
                          preferred_element_type=jnp.float32) + bamp_ref[...]
        h = jax.nn.relu(jnp.dot(P, wg1_ref[...],
                                preferred_element_type=jnp.float32)
                        + bg1c_ref[...])
        cm = jnp.dot(xb * sb, wdiff_ref[...],
                     preferred_element_type=jnp.float32) \
            + od * bdiff_ref[...]
        h_div = h + cm
        g = jax.nn.relu(
            jnp.dot(h_align, w1a_ref[...], preferred_element_type=jnp.float32)
            + jnp.dot(h_div, w1b_ref[...], preferred_element_type=jnp.float32)
            + bgate_ref[...])
        alpha = jax.nn.sigmoid(
            jnp.sum(g * w2_ref[...], axis=1, keepdims=True) + bg2_ref[...])
        out_ref[...] = alpha * h_align + (1.0 - alpha) * h_div

    wspec = pl.BlockSpec((D, D), lambda i: (0, 0))
    bspec = pl.BlockSpec((1, D), lambda i: (0, 0))
    return pl.pallas_call(
        body,
        grid=(grid,),
        in_specs=[
            pl.BlockSpec((blk, D), lambda i: (i, 0)),
            pl.BlockSpec((blk, D), lambda i: (i, 0)),
            pl.BlockSpec((blk, D), lambda i: (i, 0)),
            pl.BlockSpec((blk, 1), lambda i: (i, 0)),
            pl.BlockSpec((blk, 1), lambda i: (i, 0)),
            wspec, bspec, wspec, bspec, wspec, bspec,
            wspec, wspec, bspec, bspec,
            pl.BlockSpec((1, 1), lambda i: (0, 0)),
        ],
        out_specs=pl.BlockSpec((blk, D), lambda i: (i, 0)),
        out_shape=jax.ShapeDtypeStruct((NPAD, D), jnp.float32),
    )(x_pad, U, S, dinv, odcol, W_amp, b_amp, W_g1, b_g1, W_diff, b_diff,
      W1a, W1b, bg1, w2row, bg2)


def kernel(x, edge_index, W_amp, b_amp, W_g1, b_g1, W_diff, b_diff,
           Wg1, bg1, Wg2, bg2):
    n, d = x.shape
    e = edge_index.shape[1]
    src = edge_index[0].astype(jnp.int32)
    dst = edge_index[1].astype(jnp.int32)

    ng = -(-e // (N_SUB * GROUP))
    if ng % 2:
        ng += 1

    x_pad = jnp.zeros((NPAD, D), jnp.float32).at[:n].set(x)
    zeros = jnp.zeros((NPAD, D), jnp.float32)

    (hist,) = _hist_pass(e // NW)(_prep_hist(dst), _prep_hist(src))

    dinv, odcol, y_pad = _tc_norm(hist, x_pad, 2048)

    table = jnp.concatenate([x_pad, y_pad], axis=0)
    (acc,) = _su_pass(ng)(table, _prep_su(src, dst, ng), zeros)

    out_pad = _tc_fuse(
        x_pad, acc[1], acc[0], dinv, odcol,
        W_amp, b_amp.reshape(1, D), W_g1, b_g1.reshape(1, D),
        W_diff, b_diff.reshape(1, D),
        Wg1[:D], Wg1[D:], bg1.reshape(1, D),
        Wg2.reshape(1, D), bg2.reshape(1, 1), 2048)
    return out_pad[:n]


# speedup vs baseline: 35.7158x; 1.5063x over previous
import functools

import jax
import jax.numpy as jnp
from jax import lax
from jax.experimental import pallas as pl
from jax.experimental.pallas import tpu as pltpu
from jax.experimental.pallas import tpu_sc as plsc

N_NODES = 10000
D = 128
N_CORES = 2
N_SUB = 16
NW = N_CORES * N_SUB
GROUP = 80
DEPTH = 4
NPAD = 10240
ROWS_PER_TILE = NPAD // N_SUB
PAD_NODE = N_NODES

def _mesh():
    return plsc.VectorSubcoreMesh(
        core_axis_name="c", subcore_axis_name="s",
        num_cores=N_CORES, num_subcores=N_SUB)


def _hist_pass(lh):
    out_type = [jax.ShapeDtypeStruct((NW, 2, NPAD), jnp.float32)]
    scratch = [
        pltpu.VMEM((lh,), jnp.int32),
        pltpu.VMEM((lh,), jnp.int32),
        pltpu.VMEM((NPAD,), jnp.float32),
        pltpu.VMEM((NPAD,), jnp.float32),
    ]

    def body(didx_ref, sidx_ref, hist_out, didx_v, sidx_v, deg_loc, od_loc):
        c = lax.axis_index("c")
        s = lax.axis_index("s")
        w = c * N_SUB + s

        pltpu.sync_copy(didx_ref.at[w], didx_v)
        pltpu.sync_copy(sidx_ref.at[w], sidx_v)

        zero16 = jnp.zeros((16,), jnp.float32)

        @pl.loop(0, NPAD // 16)
        def _zero_hist(i):
            deg_loc[pl.ds(i * 16, 16)] = zero16
            od_loc[pl.ds(i * 16, 16)] = zero16

        ones16 = jnp.ones((16,), jnp.float32)

        @pl.loop(0, lh // 16)
        def _hist(j):
            dv = didx_v[pl.ds(j * 16, 16)]
            plsc.addupdate_scatter(deg_loc, [dv], ones16)
            sv = sidx_v[pl.ds(j * 16, 16)]
            plsc.addupdate_scatter(od_loc, [sv], ones16)

        pltpu.sync_copy(deg_loc, hist_out.at[w, 0])
        pltpu.sync_copy(od_loc, hist_out.at[w, 1])

    return pl.kernel(
        body, out_type=out_type, mesh=_mesh(), scratch_types=scratch,
        compiler_params=pltpu.CompilerParams(needs_layout_passes=False))


def _su_pass(ng):
    out_type = [jax.ShapeDtypeStruct((N_CORES, NPAD, D), jnp.float32)]

    scratch = [
        pltpu.VMEM((DEPTH, 2, GROUP), jnp.int32),
    ] + [pltpu.VMEM((GROUP, D), jnp.float32) for _ in range(DEPTH)] + [
        pltpu.VMEM_SHARED((NPAD, D), jnp.float32),
    ] + [pltpu.SemaphoreType.DMA] * (2 * DEPTH)

    def body(table_ref, idx_ref, zeros_ref, acc_out, idx_buf, *rest):
        bufs = rest[:DEPTH]
        acc_sc = rest[DEPTH]
        isems = rest[DEPTH + 1:DEPTH + 1 + DEPTH]
        rsems = rest[DEPTH + 1 + DEPTH:]
        c = lax.axis_index("c")
        s = lax.axis_index("s")

        pltpu.sync_copy(zeros_ref.at[pl.ds(s * ROWS_PER_TILE, ROWS_PER_TILE)],
                        acc_sc.at[pl.ds(s * ROWS_PER_TILE, ROWS_PER_TILE)])

        plsc.subcore_barrier()

        for k in range(DEPTH - 1):
            pltpu.sync_copy(idx_ref.at[c, s, k], idx_buf.at[k])
        for k in range(DEPTH - 1):
            pltpu.async_copy(table_ref.at[idx_buf.at[k, 0]],
                             bufs[k], rsems[k])
        pltpu.async_copy(idx_ref.at[c, s, DEPTH - 1],
                         idx_buf.at[DEPTH - 1], isems[DEPTH - 1])

        @pl.loop(0, ng)
        def _main(g):
            for b in range(DEPTH):
                @pl.when(lax.rem(g, DEPTH) == b)
                def _step():
                    bn = (b + DEPTH - 1) % DEPTH

                    pltpu.make_async_copy(table_ref.at[idx_buf.at[b, 0]],
                                          bufs[b], rsems[b]).wait()
                    pltpu.sync_copy(bufs[b], acc_sc.at[idx_buf.at[b, 1]],
                                    add=True)

                    @pl.when(g + DEPTH < ng)
                    def _prefetch_idx():
                        pltpu.async_copy(idx_ref.at[c, s, g + DEPTH],
                                         idx_buf.at[b], isems[b])

                    @pl.when(g + DEPTH - 1 < ng)
                    def _start_gather():
                        pltpu.make_async_copy(idx_ref.at[c, s, g + DEPTH - 1],
                                              idx_buf.at[bn],
                                              isems[bn]).wait()
                        pltpu.async_copy(table_ref.at[idx_buf.at[bn, 0]],
                                         bufs[bn], rsems[bn])

        plsc.subcore_barrier()
        pltpu.sync_copy(acc_sc.at[pl.ds(s * ROWS_PER_TILE, ROWS_PER_TILE)],
                        acc_out.at[c, pl.ds(s * ROWS_PER_TILE, ROWS_PER_TILE)])

    return pl.kernel(
        body, out_type=out_type, mesh=_mesh(), scratch_types=scratch,
        compiler_params=pltpu.CompilerParams(needs_layout_passes=False))


def _prep_hist(idx):
    return idx.reshape(NW, idx.shape[0] // NW)


def _prep_su(src, dst, ng):
    total = N_SUB * ng * GROUP
    pad = total - src.shape[0]

    def shape(a, padval):
        a = jnp.concatenate([a, jnp.full((pad,), padval, jnp.int32)])
        return a.reshape(N_SUB, ng, GROUP)

    c0 = jnp.stack([shape(dst, PAD_NODE), shape(src, PAD_NODE)], axis=2)
    c1 = jnp.stack([shape(src + NPAD, NPAD + PAD_NODE),
                    shape(dst, PAD_NODE)], axis=2)
    return jnp.stack([c0, c1], axis=0)


def _tc_norm(hist, x_pad, blk):
    grid = NPAD // blk

    def body(hist_ref, x_ref, dinv_ref, od_ref, y_ref):
        hsum = jnp.sum(hist_ref[...], axis=0)
        dinv = lax.rsqrt(hsum[0:1, :] + 1.0)
        od = hsum[1:2, :]
        dinv_col = jnp.transpose(dinv)
        od_col = jnp.transpose(od)
        dinv_ref[...] = dinv_col
        od_ref[...] = od_col
        y_ref[...] = dinv_col * x_ref[...]

    return pl.pallas_call(
        body,
        grid=(grid,),
        in_specs=[
            pl.BlockSpec((NW, 2, blk), lambda i: (0, 0, i)),
            pl.BlockSpec((blk, D), lambda i: (i, 0)),
        ],
        out_specs=[
            pl.BlockSpec((blk, 1), lambda i: (i, 0)),
            pl.BlockSpec((blk, 1), lambda i: (i, 0)),
            pl.BlockSpec((blk, D), lambda i: (i, 0)),
        ],
        out_shape=[
            jax.ShapeDtypeStruct((NPAD, 1), jnp.float32),
            jax.ShapeDtypeStruct((NPAD, 1), jnp.float32),
            jax.ShapeDtypeStruct((NPAD, D), jnp.float32),
        ],
    )(hist, x_pad)


def _tc_fuse(x_pad, U, S, dinv, odcol, W_amp, b_amp, W_g1, b_g1,
             W_diff, b_diff, W1a, W1b, bg1, w2row, bg2, blk):
    grid = NPAD // blk

    def body(x_ref, u_ref, s_ref, dinv_ref, od_ref,
             wamp_ref, bamp_ref, wg1_ref, bg1c_ref, wdiff_ref, bdiff_ref,
             w1a_ref, w1b_ref, bgate_ref, w2_ref, bg2_ref, out_ref):
        xb = x_ref[...]
        ub = u_ref[...]
        sb = s_ref[...]
        db = dinv_ref[...]
        od = od_ref[...]

        P = db * ub + (db * db) * xb
        h_align = jnp.dot(P, wamp_ref[...],
                          preferred_element_type=jnp.float32) + bamp_ref[...]
        h = jax.nn.relu(jnp.dot(P, wg1_ref[...],
                                preferred_element_type=jnp.float32)
                        + bg1c_ref[...])
        cm = jnp.dot(xb * sb, wdiff_ref[...],
                     preferred_element_type=jnp.float32) \
            + od * bdiff_ref[...]
        h_div = h + cm
        g = jax.nn.relu(
            jnp.dot(h_align, w1a_ref[...], preferred_element_type=jnp.float32)
            + jnp.dot(h_div, w1b_ref[...], preferred_element_type=jnp.float32)
            + bgate_ref[...])
        alpha = jax.nn.sigmoid(
            jnp.sum(g * w2_ref[...], axis=1, keepdims=True) + bg2_ref[...])
        out_ref[...] = alpha * h_align + (1.0 - alpha) * h_div

    wspec = pl.BlockSpec((D, D), lambda i: (0, 0))
    bspec = pl.BlockSpec((1, D), lambda i: (0, 0))
    return pl.pallas_call(
        body,
        grid=(grid,),
        in_specs=[
            pl.BlockSpec((blk, D), lambda i: (i, 0)),
            pl.BlockSpec((blk, D), lambda i: (i, 0)),
            pl.BlockSpec((blk, D), lambda i: (i, 0)),
            pl.BlockSpec((blk, 1), lambda i: (i, 0)),
            pl.BlockSpec((blk, 1), lambda i: (i, 0)),
            wspec, bspec, wspec, bspec, wspec, bspec,
            wspec, wspec, bspec, bspec,
            pl.BlockSpec((1, 1), lambda i: (0, 0)),
        ],
        out_specs=pl.BlockSpec((blk, D), lambda i: (i, 0)),
        out_shape=jax.ShapeDtypeStruct((NPAD, D), jnp.float32),
    )(x_pad, U, S, dinv, odcol, W_amp, b_amp, W_g1, b_g1, W_diff, b_diff,
      W1a, W1b, bg1, w2row, bg2)


def kernel(x, edge_index, W_amp, b_amp, W_g1, b_g1, W_diff, b_diff,
           Wg1, bg1, Wg2, bg2):
    n, d = x.shape
    e = edge_index.shape[1]
    src = edge_index[0].astype(jnp.int32)
    dst = edge_index[1].astype(jnp.int32)

    ng = -(-e // (N_SUB * GROUP))
    if ng % 2:
        ng += 1

    x_pad = jnp.zeros((NPAD, D), jnp.float32).at[:n].set(x)
    zeros = jnp.zeros((NPAD, D), jnp.float32)

    (hist,) = _hist_pass(e // NW)(_prep_hist(dst), _prep_hist(src))

    dinv, odcol, y_pad = _tc_norm(hist, x_pad, 2048)

    table = jnp.concatenate([x_pad, y_pad], axis=0)
    (acc,) = _su_pass(ng)(table, _prep_su(src, dst, ng), zeros)

    out_pad = _tc_fuse(
        x_pad, acc[1], acc[0], dinv, odcol,
        W_amp, b_amp.reshape(1, D), W_g1, b_g1.reshape(1, D),
        W_diff, b_diff.reshape(1, D),
        Wg1[:D], Wg1[D:], bg1.reshape(1, D),
        Wg2.reshape(1, D), bg2.reshape(1, 1), 2048)
    return out_pad[:n]
